# baseline (speedup 1.0000x reference)
import functools

import jax
import jax.numpy as jnp
from jax import lax
from jax.experimental import pallas as pl
from jax.experimental.pallas import tpu as pltpu
from jax.experimental.pallas import tpu_sc as plsc

B = 16384
F = 26
V = 100000
D = 16
NUM = 13

NC, NS = 2, 16
NW = NC * NS
RPW = B * F // NW
SL = 128
SPC = 26
CR = SPC * SL
NSC = RPW // CR
KPW = RPW // SL


@functools.partial(
    pl.kernel,
    out_type=jax.ShapeDtypeStruct((B * F // SL, SL, D), jnp.float32),
    mesh=plsc.VectorSubcoreMesh(core_axis_name="c", subcore_axis_name="s"),
    scratch_types=[
        pltpu.VMEM((KPW, SL), jnp.int32),
        pltpu.VMEM((2, SPC, SL, D), jnp.float32),
        pltpu.SemaphoreType.DMA,
        pltpu.SemaphoreType.DMA,
    ],
    compiler_params=pltpu.CompilerParams(use_tc_tiling_on_sc=False),
)
def _emb_kernel(tab_hbm, idx_hbm, out_hbm, idx_v, gbuf, gsem, wsem):
    wid = lax.axis_index("s") * NC + lax.axis_index("c")
    k0 = wid * KPW
    pltpu.sync_copy(idx_hbm.at[wid], idx_v)

    def fire_sc(c):
        def fire(k, carry):
            pltpu.make_async_copy(
                tab_hbm.at[idx_v.at[c * SPC + k]], gbuf.at[c % 2, k], gsem
            ).start()
            return carry

        lax.fori_loop(0, SPC, fire, 0)

    def wb_copy(c):
        return pltpu.make_async_copy(
            gbuf.at[c % 2], out_hbm.at[pl.ds(k0 + c * SPC, SPC)], wsem
        )

    fire_sc(0)

    def chunk_body(c, carry):
        @pl.when(c + 1 < NSC)
        def _():
            @pl.when(c >= 1)
            def _():
                wb_copy(c - 1).wait()

            fire_sc(c + 1)

        def drain(k, carry2):
            pltpu.make_async_copy(
                tab_hbm.at[idx_v.at[c * SPC + k]], gbuf.at[c % 2, k], gsem
            ).wait()
            return carry2

        lax.fori_loop(0, SPC, drain, 0)
        wb_copy(c).start()
        return carry

    lax.fori_loop(0, NSC, chunk_body, 0)
    wb_copy(NSC - 2).wait()
    wb_copy(NSC - 1).wait()


def kernel(indices, numerical, tables):
    tab2 = tables.reshape(F * V, D)
    flat = indices.astype(jnp.int32) + (jnp.arange(F, dtype=jnp.int32) * V)[None, :]
    idx3 = flat.reshape(NW, KPW, SL)
    emb = _emb_kernel(tab2, idx3).reshape(B, F * D)
    return jnp.concatenate([numerical, emb], axis=1)
